# Initial kernel scaffold; baseline (speedup 1.0000x reference)
import jax
import jax.numpy as jnp
from jax.experimental import pallas as pl


def kernel(h1, h2, h3, h4, up_2_0, down_2_0, up_2_1, down_2_1, up_3_0, down_3_0, up_3_1, down_3_1, up_4_0, down_4_0, up_4_1, down_4_1, W_up_2, b_up_2, W_up_3, b_up_3, W_up_4, b_up_4, W_down_1, b_down_1, W_down_2, b_down_2, W_down_3, b_down_3):
    raise NotImplementedError("write your pallas kernel here")


# speedup vs baseline: 1.0989x; 1.0989x over previous
import functools

import jax
import jax.numpy as jnp
from jax.experimental import pallas as pl

N = 100000
U = 64
BLK = 2000


def _mlp3_body(h_ref, a0_ref, a1_ref, w_ref, b_ref, o_ref):
    z = (jnp.dot(h_ref[...], w_ref[0:U, :], preferred_element_type=jnp.float32)
         + jnp.dot(a0_ref[...], w_ref[U:2 * U, :], preferred_element_type=jnp.float32)
         + jnp.dot(a1_ref[...], w_ref[2 * U:3 * U, :], preferred_element_type=jnp.float32))
    o_ref[...] = jnp.tanh(z + b_ref[...])


def _mlp2_body(h_ref, a0_ref, w_ref, b_ref, o_ref):
    z = (jnp.dot(h_ref[...], w_ref[0:U, :], preferred_element_type=jnp.float32)
         + jnp.dot(a0_ref[...], w_ref[U:2 * U, :], preferred_element_type=jnp.float32))
    o_ref[...] = jnp.tanh(z + b_ref[...])


@functools.partial(jax.jit, static_argnums=())
def _mlp3(h, a0, a1, wt, b):
    grid = (N // BLK,)
    return pl.pallas_call(
        _mlp3_body,
        grid=grid,
        in_specs=[
            pl.BlockSpec((BLK, U), lambda i: (i, 0)),
            pl.BlockSpec((BLK, U), lambda i: (i, 0)),
            pl.BlockSpec((BLK, U), lambda i: (i, 0)),
            pl.BlockSpec((3 * U, U), lambda i: (0, 0)),
            pl.BlockSpec((1, U), lambda i: (0, 0)),
        ],
        out_specs=pl.BlockSpec((BLK, U), lambda i: (i, 0)),
        out_shape=jax.ShapeDtypeStruct((N, U), jnp.float32),
    )(h, a0, a1, wt, b)


def _mlp2(h, a0, wt, b):
    grid = (N // BLK,)
    return pl.pallas_call(
        _mlp2_body,
        grid=grid,
        in_specs=[
            pl.BlockSpec((BLK, U), lambda i: (i, 0)),
            pl.BlockSpec((BLK, U), lambda i: (i, 0)),
            pl.BlockSpec((2 * U, U), lambda i: (0, 0)),
            pl.BlockSpec((1, U), lambda i: (0, 0)),
        ],
        out_specs=pl.BlockSpec((BLK, U), lambda i: (i, 0)),
        out_shape=jax.ShapeDtypeStruct((N, U), jnp.float32),
    )(h, a0, wt, b)


def kernel(h1, h2, h3, h4, up_2_0, down_2_0, up_2_1, down_2_1, up_3_0, down_3_0, up_3_1, down_3_1, up_4_0, down_4_0, up_4_1, down_4_1, W_up_2, b_up_2, W_up_3, b_up_3, W_up_4, b_up_4, W_down_1, b_down_1, W_down_2, b_down_2, W_down_3, b_down_3):
    kw = dict(locals())
    h = {1: h1, 2: h2, 3: h3, 4: h4}
    for idx in (2, 3, 4):
        aggs = []
        for pos in (0, 1):
            e = kw["up_%d_%d" % (idx, pos)]
            msg = h[idx - 1][e[0]]
            aggs.append(jax.ops.segment_sum(msg, e[1], num_segments=N))
        wt = kw["W_up_%d" % idx].T
        b = kw["b_up_%d" % idx].reshape(1, U)
        h[idx] = _mlp3(h[idx], aggs[0], aggs[1], wt, b)
    for idx in (4, 3, 2):
        hd = jnp.zeros((N, U), dtype=jnp.float32)
        for pos in (0, 1):
            e = kw["down_%d_%d" % (idx, pos)]
            hd = hd + jax.ops.segment_sum(h[idx][e[0]], e[1], num_segments=N)
        wt = kw["W_down_%d" % (idx - 1)].T
        b = kw["b_down_%d" % (idx - 1)].reshape(1, U)
        h[idx - 1] = _mlp2(h[idx - 1], hd, wt, b)
    return jnp.stack([h[1], h[2], h[3], h[4]])


# speedup vs baseline: 2.0499x; 1.8654x over previous
import functools

import jax
import jax.numpy as jnp
from jax import lax
from jax.experimental import pallas as pl
from jax.experimental.pallas import tpu as pltpu
from jax.experimental.pallas import tpu_sc as plsc

N = 100000
E = 500000
U = 64

CW = 16
NCH = U // CW
TILES = 16
IB = 128
NB = 8
WIN = IB * NB
NWIN = 500
E_PAD = NWIN * WIN
NPAD = 100096
RPT = NPAD // TILES
ZR = 272


@functools.cache
def _make_seg1():
    mesh = plsc.VectorSubcoreMesh(core_axis_name="c", subcore_axis_name="s",
                                  num_cores=1, num_subcores=TILES)

    @functools.partial(
        pl.kernel,
        out_type=jax.ShapeDtypeStruct((NCH, NPAD, CW), jnp.float32),
        mesh=mesh,
        scratch_types=[
            pltpu.VMEM((NB, IB), jnp.int32),
            pltpu.VMEM((NB, IB), jnp.int32),
            pltpu.VMEM((WIN, CW), jnp.float32),
            pltpu.VMEM((ZR, CW), jnp.float32),
            pltpu.VMEM_SHARED((NPAD, CW), jnp.float32),
            pltpu.SemaphoreType.DMA,
            pltpu.SemaphoreType.DMA,
        ],
        compiler_params=pltpu.CompilerParams(use_tc_tiling_on_sc=False),
    )
    def _seg1(tab, src3, dst3, out, srcv, dstv, rows, zbuf, acc, sem0, sem1):
        s = lax.axis_index("s")
        sems = (sem0, sem1)

        @pl.loop(0, ZR, unroll=8)
        def _zero_zbuf(i):
            zbuf[i] = jnp.zeros((CW,), jnp.float32)

        nwin = (NWIN - s + TILES - 1) // TILES

        for ch in range(NCH):
            for zi in range(RPT // ZR):
                pltpu.sync_copy(zbuf, acc.at[pl.ds(s * RPT + zi * ZR, ZR)])
            plsc.subcore_barrier()
            tabc = tab.at[ch]

            @pl.loop(0, nwin)
            def _window(j):
                w = s + TILES * j
                pltpu.sync_copy(src3.at[w], srcv)
                pltpu.sync_copy(dst3.at[w], dstv)
                descs = [None] * NB
                descs[0] = pltpu.async_copy(
                    tabc.at[srcv.at[0]], rows.at[pl.ds(0, IB)], sems[0])
                for i in range(NB):
                    if i + 1 < NB:
                        descs[i + 1] = pltpu.async_copy(
                            tabc.at[srcv.at[i + 1]],
                            rows.at[pl.ds((i + 1) * IB, IB)],
                            sems[(i + 1) % 2])
                    descs[i].wait()
                    pltpu.sync_copy(rows.at[pl.ds(i * IB, IB)],
                                    acc.at[dstv.at[i]], add=True)

            plsc.subcore_barrier()
            pltpu.sync_copy(acc.at[pl.ds(s * RPT, RPT)],
                            out.at[ch].at[pl.ds(s * RPT, RPT)])

    return _seg1


def _pad_edges(e):
    npad = E_PAD - E
    src = jnp.concatenate([e[0], jnp.arange(npad, dtype=jnp.int32) % WIN])
    dst = jnp.concatenate(
        [e[1], N + (jnp.arange(npad, dtype=jnp.int32) % 64)])
    return src.reshape(NWIN, NB, IB), dst.reshape(NWIN, NB, IB)


def _segsum1(tab, e):
    src3, dst3 = _pad_edges(e)
    return _make_seg1()(tab, src3, dst3)


BLK = 2000
_GRID = (N // BLK,)


def _bspec_h(i):
    return (i, 0)


def _bspec_w(i):
    return (0, 0)


def _bspec_g(i):
    return (0, i, 0)


_H_SPEC = pl.BlockSpec((BLK, U), _bspec_h)
_G_SPEC = pl.BlockSpec((NCH, BLK, CW), _bspec_g)
_B_SPEC = pl.BlockSpec((1, U), _bspec_w)


def _wspec(k):
    return pl.BlockSpec((k, U), _bspec_w)


def _dotp(x, w):
    return jnp.dot(x, w, preferred_element_type=jnp.float32)


def _chunks(x, w_ref, lo):
    return [_dotp(x, w_ref[lo:lo + U, CW * ch:CW * (ch + 1)]) for ch in range(NCH)]


def _cat2(a_ref, b_ref):
    return jnp.concatenate(
        [a_ref[ch] + b_ref[ch] for ch in range(NCH)], axis=-1)


def _k0_body(h1, h2, h3, h4, wu2, wu3, wu4, wd1, b2, b3, b4, bd1,
             ga2, gb2, u2, u3, u4, v1):
    x1 = h1[...]
    for ch, g in enumerate(_chunks(x1, wu2, U)):
        ga2[ch] = g
    for ch, g in enumerate(_chunks(x1, wu2, 2 * U)):
        gb2[ch] = g
    u2[...] = _dotp(h2[...], wu2[0:U, :]) + b2[...]
    u3[...] = _dotp(h3[...], wu3[0:U, :]) + b3[...]
    u4[...] = _dotp(h4[...], wu4[0:U, :]) + b4[...]
    v1[...] = _dotp(x1, wd1[0:U, :]) + bd1[...]


def _k0(h1, h2, h3, h4, wu2, wu3, wu4, wd1, b2, b3, b4, bd1):
    hs = jax.ShapeDtypeStruct((N, U), jnp.float32)
    gs = jax.ShapeDtypeStruct((NCH, N, CW), jnp.float32)
    return pl.pallas_call(
        _k0_body,
        grid=_GRID,
        in_specs=[_H_SPEC] * 4 + [_wspec(3 * U)] * 3 + [_wspec(2 * U)] + [_B_SPEC] * 4,
        out_specs=[_G_SPEC, _G_SPEC, _H_SPEC, _H_SPEC, _H_SPEC, _H_SPEC],
        out_shape=[gs, gs, hs, hs, hs, hs],
    )(h1, h2, h3, h4, wu2, wu3, wu4, wd1, b2, b3, b4, bd1)


def _ku_body(u, acca, accb, wn, wd, bd, ga, gb, v):
    hnew = jnp.tanh(u[...] + _cat2(acca, accb))
    for ch, g in enumerate(_chunks(hnew, wn, U)):
        ga[ch] = g
    for ch, g in enumerate(_chunks(hnew, wn, 2 * U)):
        gb[ch] = g
    v[...] = _dotp(hnew, wd[0:U, :]) + bd[...]


def _ku(u, acca, accb, wn, wd, bd):
    hs = jax.ShapeDtypeStruct((N, U), jnp.float32)
    gs = jax.ShapeDtypeStruct((NCH, N, CW), jnp.float32)
    return pl.pallas_call(
        _ku_body,
        grid=_GRID,
        in_specs=[_H_SPEC, _G_SPEC, _G_SPEC, _wspec(3 * U), _wspec(2 * U), _B_SPEC],
        out_specs=[_G_SPEC, _G_SPEC, _H_SPEC],
        out_shape=[gs, gs, hs],
    )(u, acca, accb, wn, wd, bd)


def _kp_body(u, acca, accb, wd, hf, gd):
    h = jnp.tanh(u[...] + _cat2(acca, accb))
    hf[...] = h
    for ch, g in enumerate(_chunks(h, wd, U)):
        gd[ch] = g


def _kp(u, acca, accb, wd):
    hs = jax.ShapeDtypeStruct((N, U), jnp.float32)
    gs = jax.ShapeDtypeStruct((NCH, N, CW), jnp.float32)
    return pl.pallas_call(
        _kp_body,
        grid=_GRID,
        in_specs=[_H_SPEC, _G_SPEC, _G_SPEC, _wspec(2 * U)],
        out_specs=[_H_SPEC, _G_SPEC],
        out_shape=[hs, gs],
    )(u, acca, accb, wd)


def _kf_body(v, acca, accb, hf):
    hf[...] = jnp.tanh(v[...] + _cat2(acca, accb))


def _kf(v, acca, accb):
    hs = jax.ShapeDtypeStruct((N, U), jnp.float32)
    return pl.pallas_call(
        _kf_body,
        grid=_GRID,
        in_specs=[_H_SPEC, _G_SPEC, _G_SPEC],
        out_specs=_H_SPEC,
        out_shape=hs,
    )(v, acca, accb)


def kernel(h1, h2, h3, h4, up_2_0, down_2_0, up_2_1, down_2_1, up_3_0, down_3_0, up_3_1, down_3_1, up_4_0, down_4_0, up_4_1, down_4_1, W_up_2, b_up_2, W_up_3, b_up_3, W_up_4, b_up_4, W_down_1, b_down_1, W_down_2, b_down_2, W_down_3, b_down_3):
    wu2, wu3, wu4 = W_up_2.T, W_up_3.T, W_up_4.T
    wd1, wd2, wd3 = W_down_1.T, W_down_2.T, W_down_3.T
    b2, b3, b4 = (b.reshape(1, U) for b in (b_up_2, b_up_3, b_up_4))
    bd1, bd2, bd3 = (b.reshape(1, U) for b in (b_down_1, b_down_2, b_down_3))

    ga2, gb2, u2, u3, u4, v1 = _k0(h1, h2, h3, h4, wu2, wu3, wu4, wd1,
                                   b2, b3, b4, bd1)
    a2, b2_ = _segsum1(ga2, up_2_0), _segsum1(gb2, up_2_1)
    ga3, gb3, v2 = _ku(u2, a2, b2_, wu3, wd2, bd2)
    a3, b3_ = _segsum1(ga3, up_3_0), _segsum1(gb3, up_3_1)
    ga4, gb4, v3 = _ku(u3, a3, b3_, wu4, wd3, bd3)
    a4, b4_ = _segsum1(ga4, up_4_0), _segsum1(gb4, up_4_1)
    h4f, gd4 = _kp(u4, a4, b4_, wd3)
    d4a, d4b = _segsum1(gd4, down_4_0), _segsum1(gd4, down_4_1)
    h3f, gd3 = _kp(v3, d4a, d4b, wd2)
    d3a, d3b = _segsum1(gd3, down_3_0), _segsum1(gd3, down_3_1)
    h2f, gd2 = _kp(v2, d3a, d3b, wd1)
    d2a, d2b = _segsum1(gd2, down_2_0), _segsum1(gd2, down_2_1)
    h1f = _kf(v1, d2a, d2b)
    return jnp.stack([h1f, h2f, h3f, h4f])


# speedup vs baseline: 3.0448x; 1.4853x over previous
import functools

import jax
import jax.numpy as jnp
from jax import lax
from jax.experimental import pallas as pl
from jax.experimental.pallas import tpu as pltpu
from jax.experimental.pallas import tpu_sc as plsc

N = 100000
E = 500000
U = 64

CW = 16
NCH = U // CW
TILES = 16
IB = 128
NB = 2
WIN = IB * NB
NWIN = 2000
GRP = 4
WPT = NWIN // TILES
E_PAD = NWIN * WIN
NPAD = 100096
RPT = NPAD // TILES
ZR = 272


@functools.cache
def _make_seg1():
    mesh = plsc.VectorSubcoreMesh(core_axis_name="c", subcore_axis_name="s",
                                  num_cores=1, num_subcores=TILES)

    @functools.partial(
        pl.kernel,
        out_type=jax.ShapeDtypeStruct((NCH, NPAD, CW), jnp.float32),
        mesh=mesh,
        scratch_types=(
            [pltpu.VMEM((NB, IB), jnp.int32)] * GRP
            + [pltpu.VMEM((NB, IB), jnp.int32)] * GRP
            + [pltpu.VMEM((WIN, CW), jnp.float32)] * GRP
            + [pltpu.VMEM((ZR, CW), jnp.float32),
               pltpu.VMEM_SHARED((NPAD, CW), jnp.float32)]
            + [pltpu.SemaphoreType.DMA] * (2 * GRP + 1)
        ),
        compiler_params=pltpu.CompilerParams(use_tc_tiling_on_sc=False),
    )
    def _seg1(tab, src3, dst3, out, *scr):
        srcv = scr[0:GRP]
        dstv = scr[GRP:2 * GRP]
        rows = scr[2 * GRP:3 * GRP]
        zbuf = scr[3 * GRP]
        acc = scr[3 * GRP + 1]
        isem = scr[3 * GRP + 2:3 * GRP + 2 + GRP]
        gsem = scr[3 * GRP + 2 + GRP:3 * GRP + 2 + 2 * GRP]
        ssem = scr[3 * GRP + 2 + 2 * GRP]
        s = lax.axis_index("s")

        @pl.loop(0, ZR, unroll=8)
        def _zero_zbuf(i):
            zbuf[i] = jnp.zeros((CW,), jnp.float32)

        def _group(t, ch, nw):
            idescs, gdescs, sdescs = [], [], []
            for b in range(nw):
                w = s + TILES * (t + b)
                idescs.append(pltpu.async_copy(src3.at[w], srcv[b], isem[b]))
                idescs.append(pltpu.async_copy(dst3.at[w], dstv[b], isem[b]))
            for b in range(nw):
                idescs[2 * b].wait()
                idescs[2 * b + 1].wait()
                gdescs.append(pltpu.async_copy(
                    tab.at[ch].at[srcv[b].at[0]],
                    rows[b].at[pl.ds(0, IB)], gsem[b]))
                gdescs.append(pltpu.async_copy(
                    tab.at[ch].at[srcv[b].at[1]],
                    rows[b].at[pl.ds(IB, IB)], gsem[b]))
            for b in range(nw):
                gdescs[2 * b].wait()
                gdescs[2 * b + 1].wait()
                sdescs.append(pltpu.async_copy(
                    rows[b].at[pl.ds(0, IB)],
                    acc.at[dstv[b].at[0]], ssem, add=True))
                sdescs.append(pltpu.async_copy(
                    rows[b].at[pl.ds(IB, IB)],
                    acc.at[dstv[b].at[1]], ssem, add=True))
            for d in sdescs:
                d.wait()

        @pl.loop(0, NCH)
        def _round(ch):
            zdescs = [pltpu.async_copy(
                zbuf, acc.at[pl.ds(s * RPT + zi * ZR, ZR)], ssem)
                for zi in range(RPT // ZR)]
            for d in zdescs:
                d.wait()
            plsc.subcore_barrier()

            @pl.loop(0, WPT - WPT % GRP, step=GRP)
            def _grp(t):
                _group(t, ch, GRP)

            if WPT % GRP:
                _group(WPT - WPT % GRP, ch, WPT % GRP)
            plsc.subcore_barrier()
            pltpu.sync_copy(acc.at[pl.ds(s * RPT, RPT)],
                            out.at[ch].at[pl.ds(s * RPT, RPT)])

    return _seg1


def _pad_edges(e):
    npad = E_PAD - E
    src = jnp.concatenate([e[0], jnp.arange(npad, dtype=jnp.int32) % WIN])
    dst = jnp.concatenate(
        [e[1], N + (jnp.arange(npad, dtype=jnp.int32) % 64)])
    return src.reshape(NWIN, NB, IB), dst.reshape(NWIN, NB, IB)


def _segsum1(tab, e):
    src3, dst3 = _pad_edges(e)
    return _make_seg1()(tab, src3, dst3)


BLK = 2000
_GRID = (N // BLK,)


def _bspec_h(i):
    return (i, 0)


def _bspec_w(i):
    return (0, 0)


def _bspec_g(i):
    return (0, i, 0)


_H_SPEC = pl.BlockSpec((BLK, U), _bspec_h)
_G_SPEC = pl.BlockSpec((NCH, BLK, CW), _bspec_g)
_B_SPEC = pl.BlockSpec((1, U), _bspec_w)


def _wspec(k):
    return pl.BlockSpec((k, U), _bspec_w)


def _dotp(x, w):
    return jnp.dot(x, w, preferred_element_type=jnp.float32)


def _chunks(x, w_ref, lo):
    return [_dotp(x, w_ref[lo:lo + U, CW * ch:CW * (ch + 1)]) for ch in range(NCH)]


def _cat2(a_ref, b_ref):
    return jnp.concatenate(
        [a_ref[ch] + b_ref[ch] for ch in range(NCH)], axis=-1)


def _k0_body(h1, h2, h3, h4, wu2, wu3, wu4, wd1, b2, b3, b4, bd1,
             ga2, gb2, u2, u3, u4, v1):
    x1 = h1[...]
    for ch, g in enumerate(_chunks(x1, wu2, U)):
        ga2[ch] = g
    for ch, g in enumerate(_chunks(x1, wu2, 2 * U)):
        gb2[ch] = g
    u2[...] = _dotp(h2[...], wu2[0:U, :]) + b2[...]
    u3[...] = _dotp(h3[...], wu3[0:U, :]) + b3[...]
    u4[...] = _dotp(h4[...], wu4[0:U, :]) + b4[...]
    v1[...] = _dotp(x1, wd1[0:U, :]) + bd1[...]


def _k0(h1, h2, h3, h4, wu2, wu3, wu4, wd1, b2, b3, b4, bd1):
    hs = jax.ShapeDtypeStruct((N, U), jnp.float32)
    gs = jax.ShapeDtypeStruct((NCH, N, CW), jnp.float32)
    return pl.pallas_call(
        _k0_body,
        grid=_GRID,
        in_specs=[_H_SPEC] * 4 + [_wspec(3 * U)] * 3 + [_wspec(2 * U)] + [_B_SPEC] * 4,
        out_specs=[_G_SPEC, _G_SPEC, _H_SPEC, _H_SPEC, _H_SPEC, _H_SPEC],
        out_shape=[gs, gs, hs, hs, hs, hs],
    )(h1, h2, h3, h4, wu2, wu3, wu4, wd1, b2, b3, b4, bd1)


def _ku_body(u, acca, accb, wn, wd, bd, ga, gb, v):
    hnew = jnp.tanh(u[...] + _cat2(acca, accb))
    for ch, g in enumerate(_chunks(hnew, wn, U)):
        ga[ch] = g
    for ch, g in enumerate(_chunks(hnew, wn, 2 * U)):
        gb[ch] = g
    v[...] = _dotp(hnew, wd[0:U, :]) + bd[...]


def _ku(u, acca, accb, wn, wd, bd):
    hs = jax.ShapeDtypeStruct((N, U), jnp.float32)
    gs = jax.ShapeDtypeStruct((NCH, N, CW), jnp.float32)
    return pl.pallas_call(
        _ku_body,
        grid=_GRID,
        in_specs=[_H_SPEC, _G_SPEC, _G_SPEC, _wspec(3 * U), _wspec(2 * U), _B_SPEC],
        out_specs=[_G_SPEC, _G_SPEC, _H_SPEC],
        out_shape=[gs, gs, hs],
    )(u, acca, accb, wn, wd, bd)


def _kp_body(u, acca, accb, wd, hf, gd):
    h = jnp.tanh(u[...] + _cat2(acca, accb))
    hf[...] = h
    for ch, g in enumerate(_chunks(h, wd, U)):
        gd[ch] = g


def _kp(u, acca, accb, wd):
    hs = jax.ShapeDtypeStruct((N, U), jnp.float32)
    gs = jax.ShapeDtypeStruct((NCH, N, CW), jnp.float32)
    return pl.pallas_call(
        _kp_body,
        grid=_GRID,
        in_specs=[_H_SPEC, _G_SPEC, _G_SPEC, _wspec(2 * U)],
        out_specs=[_H_SPEC, _G_SPEC],
        out_shape=[hs, gs],
    )(u, acca, accb, wd)


def _kf_body(v, acca, accb, hf):
    hf[...] = jnp.tanh(v[...] + _cat2(acca, accb))


def _kf(v, acca, accb):
    hs = jax.ShapeDtypeStruct((N, U), jnp.float32)
    return pl.pallas_call(
        _kf_body,
        grid=_GRID,
        in_specs=[_H_SPEC, _G_SPEC, _G_SPEC],
        out_specs=_H_SPEC,
        out_shape=hs,
    )(v, acca, accb)


def kernel(h1, h2, h3, h4, up_2_0, down_2_0, up_2_1, down_2_1, up_3_0, down_3_0, up_3_1, down_3_1, up_4_0, down_4_0, up_4_1, down_4_1, W_up_2, b_up_2, W_up_3, b_up_3, W_up_4, b_up_4, W_down_1, b_down_1, W_down_2, b_down_2, W_down_3, b_down_3):
    wu2, wu3, wu4 = W_up_2.T, W_up_3.T, W_up_4.T
    wd1, wd2, wd3 = W_down_1.T, W_down_2.T, W_down_3.T
    b2, b3, b4 = (b.reshape(1, U) for b in (b_up_2, b_up_3, b_up_4))
    bd1, bd2, bd3 = (b.reshape(1, U) for b in (b_down_1, b_down_2, b_down_3))

    ga2, gb2, u2, u3, u4, v1 = _k0(h1, h2, h3, h4, wu2, wu3, wu4, wd1,
                                   b2, b3, b4, bd1)
    a2, b2_ = _segsum1(ga2, up_2_0), _segsum1(gb2, up_2_1)
    ga3, gb3, v2 = _ku(u2, a2, b2_, wu3, wd2, bd2)
    a3, b3_ = _segsum1(ga3, up_3_0), _segsum1(gb3, up_3_1)
    ga4, gb4, v3 = _ku(u3, a3, b3_, wu4, wd3, bd3)
    a4, b4_ = _segsum1(ga4, up_4_0), _segsum1(gb4, up_4_1)
    h4f, gd4 = _kp(u4, a4, b4_, wd3)
    d4a, d4b = _segsum1(gd4, down_4_0), _segsum1(gd4, down_4_1)
    h3f, gd3 = _kp(v3, d4a, d4b, wd2)
    d3a, d3b = _segsum1(gd3, down_3_0), _segsum1(gd3, down_3_1)
    h2f, gd2 = _kp(v2, d3a, d3b, wd1)
    d2a, d2b = _segsum1(gd2, down_2_0), _segsum1(gd2, down_2_1)
    h1f = _kf(v1, d2a, d2b)
    return jnp.stack([h1f, h2f, h3f, h4f])


# speedup vs baseline: 3.9276x; 1.2899x over previous
import functools

import jax
import jax.numpy as jnp
from jax import lax
from jax.experimental import pallas as pl
from jax.experimental.pallas import tpu as pltpu
from jax.experimental.pallas import tpu_sc as plsc

N = 100000
E = 500000
U = 64

CW = 16
NCH = U // CW
TILES = 16
IB = 128
NB = 2
WIN = IB * NB
NWIN = 2000
E_PAD = NWIN * WIN
WPT = NWIN // TILES
GRP = 4
GFULL = WPT // GRP
NPAD = 100096
RPT = NPAD // TILES
ZR = 272
NT = N + 8
NT16 = NT * 8
NROWS = 8 * N


@functools.cache
def _make_seg():
    mesh = plsc.VectorSubcoreMesh(core_axis_name="c", subcore_axis_name="s",
                                  num_cores=1, num_subcores=TILES)

    @functools.partial(
        pl.kernel,
        out_type=jax.ShapeDtypeStruct((NCH, NPAD, CW), jnp.float32),
        mesh=mesh,
        scratch_types=(
            [pltpu.VMEM((GRP * NB, IB), jnp.int32)] * 2
            + [pltpu.VMEM((NB * IB, CW), jnp.float32)] * GRP
            + [pltpu.VMEM((ZR, CW), jnp.float32),
               pltpu.VMEM_SHARED((NPAD, CW), jnp.float32)]
            + [pltpu.SemaphoreType.DMA] * (GRP + 2)
        ),
        compiler_params=pltpu.CompilerParams(use_tc_tiling_on_sc=False),
    )
    def _seg(tab16, src_a, dst_a, src_b, dst_b, out, *scr):
        srcv, dstv = scr[0], scr[1]
        rows = scr[2:2 + GRP]
        zbuf = scr[2 + GRP]
        acc = scr[3 + GRP]
        isem = scr[4 + GRP]
        gsem = scr[5 + GRP:5 + 2 * GRP]
        ssem = scr[5 + 2 * GRP]
        s = lax.axis_index("s")

        @pl.loop(0, ZR, unroll=8)
        def _zero_zbuf(i):
            zbuf[i] = jnp.zeros((CW,), jnp.float32)

        def _group(tabs, src2, dst2, base, nw):
            nb = nw * NB
            i0 = pltpu.async_copy(src2.at[pl.ds(base, nb)],
                                  srcv.at[pl.ds(0, nb)], isem)
            i1 = pltpu.async_copy(dst2.at[pl.ds(base, nb)],
                                  dstv.at[pl.ds(0, nb)], isem)
            i0.wait()
            i1.wait()
            gdescs = []
            for q in range(nw):
                for k in range(NB):
                    gdescs.append(pltpu.async_copy(
                        tabs.at[srcv.at[q * NB + k]],
                        rows[q].at[pl.ds(k * IB, IB)], gsem[q]))
            sdescs = []
            for q in range(nw):
                for k in range(NB):
                    gdescs[q * NB + k].wait()
                for k in range(NB):
                    sdescs.append(pltpu.async_copy(
                        rows[q].at[pl.ds(k * IB, IB)],
                        acc.at[dstv.at[q * NB + k]], ssem, add=True))
            for d in sdescs:
                d.wait()

        @pl.loop(0, NCH)
        def _round(ch):
            zdescs = [pltpu.async_copy(
                zbuf, acc.at[pl.ds(s * RPT + zi * ZR, ZR)], ssem)
                for zi in range(RPT // ZR)]
            for d in zdescs:
                d.wait()
            plsc.subcore_barrier()
            tabs = tab16.at[pl.ds(ch, NROWS)]
            for src2, dst2 in ((src_a, dst_a), (src_b, dst_b)):
                @pl.loop(0, GFULL)
                def _g(g):
                    _group(tabs, src2, dst2, s * (WPT * NB) + g * (GRP * NB),
                           GRP)
                _group(tabs, src2, dst2, s * (WPT * NB) + GFULL * (GRP * NB),
                       WPT - GFULL * GRP)
            plsc.subcore_barrier()
            pltpu.sync_copy(acc.at[pl.ds(s * RPT, RPT)],
                            out.at[ch].at[pl.ds(s * RPT, RPT)])

    return _seg


def _pad_edges(e, half):
    npad = E_PAD - E
    ar = jnp.arange(npad, dtype=jnp.int32)
    src = jnp.concatenate([e[0] * 8 + 4 * half, (ar % 1024) * 8 + 4 * half])
    dst = jnp.concatenate([e[1], N + (ar % 64)])
    return src.reshape(NWIN * NB, IB), dst.reshape(NWIN * NB, IB)


def _segsum2(tcomb, e_a, e_b):
    tab16 = tcomb.reshape(NT16, CW)
    src_a, dst_a = _pad_edges(e_a, 0)
    src_b, dst_b = _pad_edges(e_b, 1)
    return _make_seg()(tab16, src_a, dst_a, src_b, dst_b)


BLK = 2000
_GRID = (N // BLK,)


def _bspec_h(i):
    return (i, 0)


def _bspec_w(i):
    return (0, 0)


def _bspec_g(i):
    return (0, i, 0)


_H_SPEC = pl.BlockSpec((BLK, U), _bspec_h)
_T_SPEC = pl.BlockSpec((BLK, 2 * U), _bspec_h)
_G_SPEC = pl.BlockSpec((NCH, BLK, CW), _bspec_g)
_B_SPEC = pl.BlockSpec((1, U), _bspec_w)
_TS = jax.ShapeDtypeStruct((NT, 2 * U), jnp.float32)
_HS = jax.ShapeDtypeStruct((N, U), jnp.float32)


def _wspec(k, m=U):
    return pl.BlockSpec((k, m), _bspec_w)


def _dotp(x, w):
    return jnp.dot(x, w, preferred_element_type=jnp.float32)


def _cat(a_ref):
    return jnp.concatenate([a_ref[ch] for ch in range(NCH)], axis=-1)


def _k0_body(h1, h2, h3, h4, wc2, wu2, wu3, wu4, wd1, b2, b3, b4, bd1,
             t2, u2, u3, u4, v1):
    x1 = h1[...]
    t2[...] = _dotp(x1, wc2[...])
    u2[...] = _dotp(h2[...], wu2[...]) + b2[...]
    u3[...] = _dotp(h3[...], wu3[...]) + b3[...]
    u4[...] = _dotp(h4[...], wu4[...]) + b4[...]
    v1[...] = _dotp(x1, wd1[...]) + bd1[...]


def _k0(h1, h2, h3, h4, wc2, wu2, wu3, wu4, wd1, b2, b3, b4, bd1):
    return pl.pallas_call(
        _k0_body,
        grid=_GRID,
        in_specs=[_H_SPEC] * 4 + [_wspec(U, 2 * U)] + [_wspec(U)] * 4
        + [_B_SPEC] * 4,
        out_specs=[_T_SPEC, _H_SPEC, _H_SPEC, _H_SPEC, _H_SPEC],
        out_shape=[_TS, _HS, _HS, _HS, _HS],
    )(h1, h2, h3, h4, wc2, wu2, wu3, wu4, wd1, b2, b3, b4, bd1)


def _ku_body(u, acc, wc, wd, bd, t, v):
    hnew = jnp.tanh(u[...] + _cat(acc))
    t[...] = _dotp(hnew, wc[...])
    v[...] = _dotp(hnew, wd[...]) + bd[...]


def _ku(u, acc, wc, wd, bd):
    return pl.pallas_call(
        _ku_body,
        grid=_GRID,
        in_specs=[_H_SPEC, _G_SPEC, _wspec(U, 2 * U), _wspec(U), _B_SPEC],
        out_specs=[_T_SPEC, _H_SPEC],
        out_shape=[_TS, _HS],
    )(u, acc, wc, wd, bd)


def _kp_body(u, acc, wc, hf, t):
    h = jnp.tanh(u[...] + _cat(acc))
    hf[...] = h
    t[...] = _dotp(h, wc[...])


def _kp(u, acc, wc):
    return pl.pallas_call(
        _kp_body,
        grid=_GRID,
        in_specs=[_H_SPEC, _G_SPEC, _wspec(U, 2 * U)],
        out_specs=[_H_SPEC, _T_SPEC],
        out_shape=[_HS, _TS],
    )(u, acc, wc)


def _kf_body(v, acc, hf):
    hf[...] = jnp.tanh(v[...] + _cat(acc))


def _kf(v, acc):
    return pl.pallas_call(
        _kf_body,
        grid=_GRID,
        in_specs=[_H_SPEC, _G_SPEC],
        out_specs=_H_SPEC,
        out_shape=_HS,
    )(v, acc)


def kernel(h1, h2, h3, h4, up_2_0, down_2_0, up_2_1, down_2_1, up_3_0, down_3_0, up_3_1, down_3_1, up_4_0, down_4_0, up_4_1, down_4_1, W_up_2, b_up_2, W_up_3, b_up_3, W_up_4, b_up_4, W_down_1, b_down_1, W_down_2, b_down_2, W_down_3, b_down_3):
    wu2, wu3, wu4 = W_up_2.T, W_up_3.T, W_up_4.T
    wd1, wd2, wd3 = W_down_1.T, W_down_2.T, W_down_3.T
    wc2 = jnp.concatenate([wu2[U:2 * U, :], wu2[2 * U:3 * U, :]], axis=1)
    wc3 = jnp.concatenate([wu3[U:2 * U, :], wu3[2 * U:3 * U, :]], axis=1)
    wc4 = jnp.concatenate([wu4[U:2 * U, :], wu4[2 * U:3 * U, :]], axis=1)
    wcd3 = jnp.concatenate([wd3[U:2 * U, :], wd3[U:2 * U, :]], axis=1)
    wcd2 = jnp.concatenate([wd2[U:2 * U, :], wd2[U:2 * U, :]], axis=1)
    wcd1 = jnp.concatenate([wd1[U:2 * U, :], wd1[U:2 * U, :]], axis=1)
    b2, b3, b4 = (b.reshape(1, U) for b in (b_up_2, b_up_3, b_up_4))
    bd1, bd2, bd3 = (b.reshape(1, U) for b in (b_down_1, b_down_2, b_down_3))

    t2, u2, u3, u4, v1 = _k0(h1, h2, h3, h4, wc2, wu2[0:U, :], wu3[0:U, :],
                             wu4[0:U, :], wd1[0:U, :], b2, b3, b4, bd1)
    acc2 = _segsum2(t2, up_2_0, up_2_1)
    t3, v2 = _ku(u2, acc2, wc3, wd2[0:U, :], bd2)
    acc3 = _segsum2(t3, up_3_0, up_3_1)
    t4, v3 = _ku(u3, acc3, wc4, wd3[0:U, :], bd3)
    acc4 = _segsum2(t4, up_4_0, up_4_1)
    h4f, td4 = _kp(u4, acc4, wcd3)
    accd4 = _segsum2(td4, down_4_0, down_4_1)
    h3f, td3 = _kp(v3, accd4, wcd2)
    accd3 = _segsum2(td3, down_3_0, down_3_1)
    h2f, td2 = _kp(v2, accd3, wcd1)
    accd2 = _segsum2(td2, down_2_0, down_2_1)
    h1f = _kf(v1, accd2)
    return jnp.stack([h1f, h2f, h3f, h4f])


# speedup vs baseline: 4.0857x; 1.0403x over previous
import functools

import jax
import jax.numpy as jnp
from jax import lax
from jax.experimental import pallas as pl
from jax.experimental.pallas import tpu as pltpu
from jax.experimental.pallas import tpu_sc as plsc

N = 100000
E = 500000
U = 64

CW = 16
NCH = U // CW
TILES = 16
IB = 128
NB = 2
WIN = IB * NB
NWIN = 2000
E_PAD = NWIN * WIN
WPT = NWIN // TILES
GRP = 4
GFULL = WPT // GRP
NPAD = 100096
RPT = NPAD // TILES
ZR = 272
NT = N + 8
NT16 = NT * 8
NROWS = 8 * N


@functools.cache
def _make_seg(ch0):
    mesh = plsc.VectorSubcoreMesh(core_axis_name="c", subcore_axis_name="s",
                                  num_cores=1, num_subcores=TILES)

    @functools.partial(
        pl.kernel,
        out_type=jax.ShapeDtypeStruct((2, NPAD, CW), jnp.float32),
        mesh=mesh,
        scratch_types=(
            [pltpu.VMEM((GRP * NB, IB), jnp.int32)] * 2
            + [pltpu.VMEM((NB * IB, CW), jnp.float32)] * GRP
            + [pltpu.VMEM((ZR, CW), jnp.float32),
               pltpu.VMEM_SHARED((NPAD, CW), jnp.float32)]
            + [pltpu.SemaphoreType.DMA] * (GRP + 2)
        ),
        compiler_params=pltpu.CompilerParams(use_tc_tiling_on_sc=False),
    )
    def _seg(tab16, src_a, dst_a, src_b, dst_b, out, *scr):
        srcv, dstv = scr[0], scr[1]
        rows = scr[2:2 + GRP]
        zbuf = scr[2 + GRP]
        acc = scr[3 + GRP]
        isem = scr[4 + GRP]
        gsem = scr[5 + GRP:5 + 2 * GRP]
        ssem = scr[5 + 2 * GRP]
        s = lax.axis_index("s")

        @pl.loop(0, ZR, unroll=8)
        def _zero_zbuf(i):
            zbuf[i] = jnp.zeros((CW,), jnp.float32)

        def _group(tabs, src2, dst2, base, nw):
            nb = nw * NB
            i0 = pltpu.async_copy(src2.at[pl.ds(base, nb)],
                                  srcv.at[pl.ds(0, nb)], isem)
            i1 = pltpu.async_copy(dst2.at[pl.ds(base, nb)],
                                  dstv.at[pl.ds(0, nb)], isem)
            i0.wait()
            i1.wait()
            gdescs = []
            for q in range(nw):
                for k in range(NB):
                    gdescs.append(pltpu.async_copy(
                        tabs.at[srcv.at[q * NB + k]],
                        rows[q].at[pl.ds(k * IB, IB)], gsem[q]))
            sdescs = []
            for q in range(nw):
                for k in range(NB):
                    gdescs[q * NB + k].wait()
                for k in range(NB):
                    sdescs.append(pltpu.async_copy(
                        rows[q].at[pl.ds(k * IB, IB)],
                        acc.at[dstv.at[q * NB + k]], ssem, add=True))
            for d in sdescs:
                d.wait()

        @pl.loop(0, 2)
        def _round(r):
            ch = ch0 + r
            zdescs = [pltpu.async_copy(
                zbuf, acc.at[pl.ds(s * RPT + zi * ZR, ZR)], ssem)
                for zi in range(RPT // ZR)]
            for d in zdescs:
                d.wait()
            plsc.subcore_barrier()
            tabs = tab16.at[pl.ds(ch, NROWS)]
            for src2, dst2 in ((src_a, dst_a), (src_b, dst_b)):
                @pl.loop(0, GFULL)
                def _g(g):
                    _group(tabs, src2, dst2, s * (WPT * NB) + g * (GRP * NB),
                           GRP)
                _group(tabs, src2, dst2, s * (WPT * NB) + GFULL * (GRP * NB),
                       WPT - GFULL * GRP)
            plsc.subcore_barrier()
            pltpu.sync_copy(acc.at[pl.ds(s * RPT, RPT)],
                            out.at[r].at[pl.ds(s * RPT, RPT)])

    return _seg


def _pad_edges(e, half):
    npad = E_PAD - E
    ar = jnp.arange(npad, dtype=jnp.int32)
    src = jnp.concatenate([e[0] * 8 + 4 * half, (ar % 1024) * 8 + 4 * half])
    dst = jnp.concatenate([e[1], N + (ar % 64)])
    return src.reshape(NWIN * NB, IB), dst.reshape(NWIN * NB, IB)


def _segsum2(tcomb, e_a, e_b):
    tab16 = tcomb.reshape(NT16, CW)
    src_a, dst_a = _pad_edges(e_a, 0)
    src_b, dst_b = _pad_edges(e_b, 1)
    lo = _make_seg(0)(tab16, src_a, dst_a, src_b, dst_b)
    hi = _make_seg(2)(tab16, src_a, dst_a, src_b, dst_b)
    return lo, hi


BLK = 2000
_GRID = (N // BLK,)


def _bspec_h(i):
    return (i, 0)


def _bspec_w(i):
    return (0, 0)


def _bspec_g(i):
    return (0, i, 0)


_H_SPEC = pl.BlockSpec((BLK, U), _bspec_h)
_T_SPEC = pl.BlockSpec((BLK, 2 * U), _bspec_h)
_G_SPEC = pl.BlockSpec((2, BLK, CW), _bspec_g)
_B_SPEC = pl.BlockSpec((1, U), _bspec_w)
_TS = jax.ShapeDtypeStruct((NT, 2 * U), jnp.float32)
_HS = jax.ShapeDtypeStruct((N, U), jnp.float32)


def _wspec(k, m=U):
    return pl.BlockSpec((k, m), _bspec_w)


def _dotp(x, w):
    return jnp.dot(x, w, preferred_element_type=jnp.float32)


def _cat(lo_ref, hi_ref):
    return jnp.concatenate(
        [lo_ref[0], lo_ref[1], hi_ref[0], hi_ref[1]], axis=-1)


def _k0_body(h1, h2, h3, h4, wc2, wu2, wu3, wu4, wd1, b2, b3, b4, bd1,
             t2, u2, u3, u4, v1):
    x1 = h1[...]
    t2[...] = _dotp(x1, wc2[...])
    u2[...] = _dotp(h2[...], wu2[...]) + b2[...]
    u3[...] = _dotp(h3[...], wu3[...]) + b3[...]
    u4[...] = _dotp(h4[...], wu4[...]) + b4[...]
    v1[...] = _dotp(x1, wd1[...]) + bd1[...]


def _k0(h1, h2, h3, h4, wc2, wu2, wu3, wu4, wd1, b2, b3, b4, bd1):
    return pl.pallas_call(
        _k0_body,
        grid=_GRID,
        in_specs=[_H_SPEC] * 4 + [_wspec(U, 2 * U)] + [_wspec(U)] * 4
        + [_B_SPEC] * 4,
        out_specs=[_T_SPEC, _H_SPEC, _H_SPEC, _H_SPEC, _H_SPEC],
        out_shape=[_TS, _HS, _HS, _HS, _HS],
    )(h1, h2, h3, h4, wc2, wu2, wu3, wu4, wd1, b2, b3, b4, bd1)


def _ku_body(u, acclo, acchi, wc, wd, bd, t, v):
    hnew = jnp.tanh(u[...] + _cat(acclo, acchi))
    t[...] = _dotp(hnew, wc[...])
    v[...] = _dotp(hnew, wd[...]) + bd[...]


def _ku(u, acc, wc, wd, bd):
    return pl.pallas_call(
        _ku_body,
        grid=_GRID,
        in_specs=[_H_SPEC, _G_SPEC, _G_SPEC, _wspec(U, 2 * U), _wspec(U),
                  _B_SPEC],
        out_specs=[_T_SPEC, _H_SPEC],
        out_shape=[_TS, _HS],
    )(u, acc[0], acc[1], wc, wd, bd)


def _kp_body(u, acclo, acchi, wc, hf, t):
    h = jnp.tanh(u[...] + _cat(acclo, acchi))
    hf[...] = h
    t[...] = _dotp(h, wc[...])


def _kp(u, acc, wc):
    return pl.pallas_call(
        _kp_body,
        grid=_GRID,
        in_specs=[_H_SPEC, _G_SPEC, _G_SPEC, _wspec(U, 2 * U)],
        out_specs=[_H_SPEC, _T_SPEC],
        out_shape=[_HS, _TS],
    )(u, acc[0], acc[1], wc)


def _kf_body(v, acclo, acchi, hf):
    hf[...] = jnp.tanh(v[...] + _cat(acclo, acchi))


def _kf(v, acc):
    return pl.pallas_call(
        _kf_body,
        grid=_GRID,
        in_specs=[_H_SPEC, _G_SPEC, _G_SPEC],
        out_specs=_H_SPEC,
        out_shape=_HS,
    )(v, acc[0], acc[1])


def kernel(h1, h2, h3, h4, up_2_0, down_2_0, up_2_1, down_2_1, up_3_0, down_3_0, up_3_1, down_3_1, up_4_0, down_4_0, up_4_1, down_4_1, W_up_2, b_up_2, W_up_3, b_up_3, W_up_4, b_up_4, W_down_1, b_down_1, W_down_2, b_down_2, W_down_3, b_down_3):
    wu2, wu3, wu4 = W_up_2.T, W_up_3.T, W_up_4.T
    wd1, wd2, wd3 = W_down_1.T, W_down_2.T, W_down_3.T
    wc2 = jnp.concatenate([wu2[U:2 * U, :], wu2[2 * U:3 * U, :]], axis=1)
    wc3 = jnp.concatenate([wu3[U:2 * U, :], wu3[2 * U:3 * U, :]], axis=1)
    wc4 = jnp.concatenate([wu4[U:2 * U, :], wu4[2 * U:3 * U, :]], axis=1)
    wcd3 = jnp.concatenate([wd3[U:2 * U, :], wd3[U:2 * U, :]], axis=1)
    wcd2 = jnp.concatenate([wd2[U:2 * U, :], wd2[U:2 * U, :]], axis=1)
    wcd1 = jnp.concatenate([wd1[U:2 * U, :], wd1[U:2 * U, :]], axis=1)
    b2, b3, b4 = (b.reshape(1, U) for b in (b_up_2, b_up_3, b_up_4))
    bd1, bd2, bd3 = (b.reshape(1, U) for b in (b_down_1, b_down_2, b_down_3))

    t2, u2, u3, u4, v1 = _k0(h1, h2, h3, h4, wc2, wu2[0:U, :], wu3[0:U, :],
                             wu4[0:U, :], wd1[0:U, :], b2, b3, b4, bd1)
    acc2 = _segsum2(t2, up_2_0, up_2_1)
    t3, v2 = _ku(u2, acc2, wc3, wd2[0:U, :], bd2)
    acc3 = _segsum2(t3, up_3_0, up_3_1)
    t4, v3 = _ku(u3, acc3, wc4, wd3[0:U, :], bd3)
    acc4 = _segsum2(t4, up_4_0, up_4_1)
    h4f, td4 = _kp(u4, acc4, wcd3)
    accd4 = _segsum2(td4, down_4_0, down_4_1)
    h3f, td3 = _kp(v3, accd4, wcd2)
    accd3 = _segsum2(td3, down_3_0, down_3_1)
    h2f, td2 = _kp(v2, accd3, wcd1)
    accd2 = _segsum2(td2, down_2_0, down_2_1)
    h1f = _kf(v1, accd2)
    return jnp.stack([h1f, h2f, h3f, h4f])


# speedup vs baseline: 5.2618x; 1.2879x over previous
import functools

import jax
import jax.numpy as jnp
from jax import lax
from jax.experimental import pallas as pl
from jax.experimental.pallas import tpu as pltpu
from jax.experimental.pallas import tpu_sc as plsc

N = 100000
E = 500000
U = 64

CW = 16
NCH = U // CW
TILES = 16
IB = 128
NB = 2
WIN = IB * NB
NWIN = 2000
E_PAD = NWIN * WIN
WPT = NWIN // TILES
GRP = 5
GFULL = WPT // GRP
NPAD = 100096
RPT = NPAD // TILES
ZR = 136
NT = N + 8
NT16 = NT * 8
NROWS = 8 * N


@functools.cache
def _make_seg(ch0):
    mesh = plsc.VectorSubcoreMesh(core_axis_name="c", subcore_axis_name="s",
                                  num_cores=1, num_subcores=TILES)

    @functools.partial(
        pl.kernel,
        out_type=jax.ShapeDtypeStruct((2, NPAD, CW), jnp.float32),
        mesh=mesh,
        scratch_types=(
            [pltpu.VMEM((GRP * NB, IB), jnp.int32)] * 4
            + [pltpu.VMEM((NB * IB, CW), jnp.float32)] * GRP
            + [pltpu.VMEM((ZR, CW), jnp.float32),
               pltpu.VMEM_SHARED((NPAD, CW), jnp.float32)]
            + [pltpu.SemaphoreType.DMA] * (GRP + 3)
        ),
        compiler_params=pltpu.CompilerParams(use_tc_tiling_on_sc=False),
    )
    def _seg(tab16, src_a, dst_a, src_b, dst_b, out, *scr):
        srcv0, srcv1, dstv0, dstv1 = scr[0:4]
        rows = scr[4:4 + GRP]
        zbuf = scr[4 + GRP]
        acc = scr[5 + GRP]
        isem0, isem1 = scr[6 + GRP], scr[7 + GRP]
        gsem = scr[8 + GRP:8 + 2 * GRP]
        ssem = scr[8 + 2 * GRP]
        s = lax.axis_index("s")

        @pl.loop(0, ZR, unroll=8)
        def _zero_zbuf(i):
            zbuf[i] = jnp.zeros((CW,), jnp.float32)

        def _fire_idx(src2, dst2, g, sv, dv, sem):
            base = s * (WPT * NB) + g * (GRP * NB)
            pltpu.async_copy(src2.at[pl.ds(base, GRP * NB)], sv, sem)
            pltpu.async_copy(dst2.at[pl.ds(base, GRP * NB)], dv, sem)

        def _drain_idx(src2, sv, dv, sem):
            pltpu.make_async_copy(src2.at[pl.ds(0, GRP * NB)], sv, sem).wait()
            pltpu.make_async_copy(src2.at[pl.ds(0, GRP * NB)], dv, sem).wait()

        def _process(tabs, sv, dv):
            gdescs = []
            for q in range(GRP):
                for k in range(NB):
                    gdescs.append(pltpu.async_copy(
                        tabs.at[sv.at[q * NB + k]],
                        rows[q].at[pl.ds(k * IB, IB)], gsem[q]))
            sdescs = []
            for q in range(GRP):
                for k in range(NB):
                    gdescs[q * NB + k].wait()
                for k in range(NB):
                    sdescs.append(pltpu.async_copy(
                        rows[q].at[pl.ds(k * IB, IB)],
                        acc.at[dv.at[q * NB + k]], ssem, add=True))
            for d in sdescs:
                d.wait()

        @pl.loop(0, 2)
        def _round(r):
            ch = ch0 + r
            zdescs = [pltpu.async_copy(
                zbuf, acc.at[pl.ds(s * RPT + zi * ZR, ZR)], ssem)
                for zi in range(RPT // ZR)]
            for d in zdescs:
                d.wait()
            plsc.subcore_barrier()
            tabs = tab16.at[pl.ds(ch, NROWS)]
            for src2, dst2 in ((src_a, dst_a), (src_b, dst_b)):
                _fire_idx(src2, dst2, 0, srcv0, dstv0, isem0)

                @pl.loop(0, GFULL, step=2)
                def _p(g):
                    @pl.when(g + 1 < GFULL)
                    def _pf1():
                        _fire_idx(src2, dst2, g + 1, srcv1, dstv1, isem1)
                    _drain_idx(src2, srcv0, dstv0, isem0)
                    _process(tabs, srcv0, dstv0)

                    @pl.when(g + 1 < GFULL)
                    def _half_b():
                        @pl.when(g + 2 < GFULL)
                        def _pf0():
                            _fire_idx(src2, dst2, g + 2, srcv0, dstv0, isem0)
                        _drain_idx(src2, srcv1, dstv1, isem1)
                        _process(tabs, srcv1, dstv1)
            plsc.subcore_barrier()
            pltpu.sync_copy(acc.at[pl.ds(s * RPT, RPT)],
                            out.at[r].at[pl.ds(s * RPT, RPT)])

    return _seg


def _pad_edges(e, half):
    npad = E_PAD - E
    ar = jnp.arange(npad, dtype=jnp.int32)
    src = jnp.concatenate([e[0] * 8 + 4 * half, (ar % 1024) * 8 + 4 * half])
    dst = jnp.concatenate([e[1], N + (ar % 64)])
    return src.reshape(NWIN * NB, IB), dst.reshape(NWIN * NB, IB)


def _segsum2(tcomb, e_a, e_b):
    tab16 = tcomb.reshape(NT16, CW)
    src_a, dst_a = _pad_edges(e_a, 0)
    src_b, dst_b = _pad_edges(e_b, 1)
    lo = _make_seg(0)(tab16, src_a, dst_a, src_b, dst_b)
    hi = _make_seg(2)(tab16, src_a, dst_a, src_b, dst_b)
    return lo, hi


BLK = 3200
_GRID = ((N + BLK - 1) // BLK,)


def _bspec_h(i):
    return (i, 0)


def _bspec_w(i):
    return (0, 0)


def _bspec_g(i):
    return (0, i, 0)


_H_SPEC = pl.BlockSpec((BLK, U), _bspec_h)
_T_SPEC = pl.BlockSpec((BLK, 2 * U), _bspec_h)
_G_SPEC = pl.BlockSpec((2, BLK, CW), _bspec_g)
_B_SPEC = pl.BlockSpec((1, U), _bspec_w)
_TS = jax.ShapeDtypeStruct((NT, 2 * U), jnp.float32)
_HS = jax.ShapeDtypeStruct((N, U), jnp.float32)


def _wspec(k, m=U):
    return pl.BlockSpec((k, m), _bspec_w)


def _dotp(x, w):
    return jnp.dot(x, w, preferred_element_type=jnp.float32)


def _cat(lo_ref, hi_ref):
    return jnp.concatenate(
        [lo_ref[0], lo_ref[1], hi_ref[0], hi_ref[1]], axis=-1)


def _k0_body(h1, h2, h3, h4, wc2, wu2, wu3, wu4, wd1, b2, b3, b4, bd1,
             t2, u2, u3, u4, v1):
    x1 = h1[...]
    t2[...] = _dotp(x1, wc2[...])
    u2[...] = _dotp(h2[...], wu2[...]) + b2[...]
    u3[...] = _dotp(h3[...], wu3[...]) + b3[...]
    u4[...] = _dotp(h4[...], wu4[...]) + b4[...]
    v1[...] = _dotp(x1, wd1[...]) + bd1[...]


def _k0(h1, h2, h3, h4, wc2, wu2, wu3, wu4, wd1, b2, b3, b4, bd1):
    return pl.pallas_call(
        _k0_body,
        grid=_GRID,
        in_specs=[_H_SPEC] * 4 + [_wspec(U, 2 * U)] + [_wspec(U)] * 4
        + [_B_SPEC] * 4,
        out_specs=[_T_SPEC, _H_SPEC, _H_SPEC, _H_SPEC, _H_SPEC],
        out_shape=[_TS, _HS, _HS, _HS, _HS],
    )(h1, h2, h3, h4, wc2, wu2, wu3, wu4, wd1, b2, b3, b4, bd1)


def _ku_body(u, acclo, acchi, wc, wd, bd, t, v):
    hnew = jnp.tanh(u[...] + _cat(acclo, acchi))
    t[...] = _dotp(hnew, wc[...])
    v[...] = _dotp(hnew, wd[...]) + bd[...]


def _ku(u, acc, wc, wd, bd):
    return pl.pallas_call(
        _ku_body,
        grid=_GRID,
        in_specs=[_H_SPEC, _G_SPEC, _G_SPEC, _wspec(U, 2 * U), _wspec(U),
                  _B_SPEC],
        out_specs=[_T_SPEC, _H_SPEC],
        out_shape=[_TS, _HS],
    )(u, acc[0], acc[1], wc, wd, bd)


def _kp_body(u, acclo, acchi, wc, hf, t):
    h = jnp.tanh(u[...] + _cat(acclo, acchi))
    hf[...] = h
    t[...] = _dotp(h, wc[...])


def _kp(u, acc, wc):
    return pl.pallas_call(
        _kp_body,
        grid=_GRID,
        in_specs=[_H_SPEC, _G_SPEC, _G_SPEC, _wspec(U, 2 * U)],
        out_specs=[_H_SPEC, _T_SPEC],
        out_shape=[_HS, _TS],
    )(u, acc[0], acc[1], wc)


def _kf_body(v, acclo, acchi, hf):
    hf[...] = jnp.tanh(v[...] + _cat(acclo, acchi))


def _kf(v, acc):
    return pl.pallas_call(
        _kf_body,
        grid=_GRID,
        in_specs=[_H_SPEC, _G_SPEC, _G_SPEC],
        out_specs=_H_SPEC,
        out_shape=_HS,
    )(v, acc[0], acc[1])


def kernel(h1, h2, h3, h4, up_2_0, down_2_0, up_2_1, down_2_1, up_3_0, down_3_0, up_3_1, down_3_1, up_4_0, down_4_0, up_4_1, down_4_1, W_up_2, b_up_2, W_up_3, b_up_3, W_up_4, b_up_4, W_down_1, b_down_1, W_down_2, b_down_2, W_down_3, b_down_3):
    wu2, wu3, wu4 = W_up_2.T, W_up_3.T, W_up_4.T
    wd1, wd2, wd3 = W_down_1.T, W_down_2.T, W_down_3.T
    wc2 = jnp.concatenate([wu2[U:2 * U, :], wu2[2 * U:3 * U, :]], axis=1)
    wc3 = jnp.concatenate([wu3[U:2 * U, :], wu3[2 * U:3 * U, :]], axis=1)
    wc4 = jnp.concatenate([wu4[U:2 * U, :], wu4[2 * U:3 * U, :]], axis=1)
    wcd3 = jnp.concatenate([wd3[U:2 * U, :], wd3[U:2 * U, :]], axis=1)
    wcd2 = jnp.concatenate([wd2[U:2 * U, :], wd2[U:2 * U, :]], axis=1)
    wcd1 = jnp.concatenate([wd1[U:2 * U, :], wd1[U:2 * U, :]], axis=1)
    b2, b3, b4 = (b.reshape(1, U) for b in (b_up_2, b_up_3, b_up_4))
    bd1, bd2, bd3 = (b.reshape(1, U) for b in (b_down_1, b_down_2, b_down_3))

    t2, u2, u3, u4, v1 = _k0(h1, h2, h3, h4, wc2, wu2[0:U, :], wu3[0:U, :],
                             wu4[0:U, :], wd1[0:U, :], b2, b3, b4, bd1)
    acc2 = _segsum2(t2, up_2_0, up_2_1)
    t3, v2 = _ku(u2, acc2, wc3, wd2[0:U, :], bd2)
    acc3 = _segsum2(t3, up_3_0, up_3_1)
    t4, v3 = _ku(u3, acc3, wc4, wd3[0:U, :], bd3)
    acc4 = _segsum2(t4, up_4_0, up_4_1)
    h4f, td4 = _kp(u4, acc4, wcd3)
    accd4 = _segsum2(td4, down_4_0, down_4_1)
    h3f, td3 = _kp(v3, accd4, wcd2)
    accd3 = _segsum2(td3, down_3_0, down_3_1)
    h2f, td2 = _kp(v2, accd3, wcd1)
    accd2 = _segsum2(td2, down_2_0, down_2_1)
    h1f = _kf(v1, accd2)
    return jnp.stack([h1f, h2f, h3f, h4f])


# speedup vs baseline: 5.2680x; 1.0012x over previous
import functools

import jax
import jax.numpy as jnp
from jax import lax
from jax.experimental import pallas as pl
from jax.experimental.pallas import tpu as pltpu
from jax.experimental.pallas import tpu_sc as plsc

N = 100000
E = 500000
U = 64

CW = 16
NCH = U // CW
TILES = 16
IB = 128
NB = 2
WIN = IB * NB
NWIN = 2000
E_PAD = NWIN * WIN
WPT = NWIN // TILES
GRP = 5
GFULL = WPT // GRP
NPAD = 100096
RPT = NPAD // TILES
ZR = 136
NT = N + 8
NT16 = NT * 8
NROWS = 8 * N


@functools.cache
def _make_seg(ch0):
    mesh = plsc.VectorSubcoreMesh(core_axis_name="c", subcore_axis_name="s",
                                  num_cores=1, num_subcores=TILES)

    @functools.partial(
        pl.kernel,
        out_type=jax.ShapeDtypeStruct((2, NPAD, CW), jnp.float32),
        mesh=mesh,
        scratch_types=(
            [pltpu.VMEM((GRP * NB, IB), jnp.int32)] * 4
            + [pltpu.VMEM((NB * IB, CW), jnp.float32)] * GRP
            + [pltpu.VMEM((ZR, CW), jnp.float32),
               pltpu.VMEM_SHARED((NPAD, CW), jnp.float32)]
            + [pltpu.SemaphoreType.DMA] * (GRP + 3)
        ),
        compiler_params=pltpu.CompilerParams(use_tc_tiling_on_sc=False),
    )
    def _seg(tab16, src_a, dst_a, src_b, dst_b, out, *scr):
        srcv0, srcv1, dstv0, dstv1 = scr[0:4]
        rows = scr[4:4 + GRP]
        zbuf = scr[4 + GRP]
        acc = scr[5 + GRP]
        isem0, isem1 = scr[6 + GRP], scr[7 + GRP]
        gsem = scr[8 + GRP:8 + 2 * GRP]
        ssem = scr[8 + 2 * GRP]
        s = lax.axis_index("s")

        @pl.loop(0, ZR, unroll=8)
        def _zero_zbuf(i):
            zbuf[i] = jnp.zeros((CW,), jnp.float32)

        def _fire_idx(src2, dst2, g, sv, dv, sem):
            base = s * (WPT * NB) + g * (GRP * NB)
            pltpu.async_copy(src2.at[pl.ds(base, GRP * NB)], sv, sem)
            pltpu.async_copy(dst2.at[pl.ds(base, GRP * NB)], dv, sem)

        def _drain_idx(src2, sv, dv, sem):
            pltpu.make_async_copy(src2.at[pl.ds(0, GRP * NB)], sv, sem).wait()
            pltpu.make_async_copy(src2.at[pl.ds(0, GRP * NB)], dv, sem).wait()

        def _process(tabs, sv, dv):
            gdescs = []
            for q in range(GRP):
                for k in range(NB):
                    gdescs.append(pltpu.async_copy(
                        tabs.at[sv.at[q * NB + k]],
                        rows[q].at[pl.ds(k * IB, IB)], gsem[q]))
            sdescs = []
            for q in range(GRP):
                for k in range(NB):
                    gdescs[q * NB + k].wait()
                for k in range(NB):
                    sdescs.append(pltpu.async_copy(
                        rows[q].at[pl.ds(k * IB, IB)],
                        acc.at[dv.at[q * NB + k]], ssem, add=True))
            for d in sdescs:
                d.wait()

        @pl.loop(0, 2)
        def _round(r):
            ch = ch0 + r
            zdescs = [pltpu.async_copy(
                zbuf, acc.at[pl.ds(s * RPT + zi * ZR, ZR)], ssem)
                for zi in range(RPT // ZR)]
            for d in zdescs:
                d.wait()
            plsc.subcore_barrier()
            tabs = tab16.at[pl.ds(ch, NROWS)]
            for src2, dst2 in ((src_a, dst_a), (src_b, dst_b)):
                _fire_idx(src2, dst2, 0, srcv0, dstv0, isem0)

                @pl.loop(0, GFULL, step=2)
                def _p(g):
                    @pl.when(g + 1 < GFULL)
                    def _pf1():
                        _fire_idx(src2, dst2, g + 1, srcv1, dstv1, isem1)
                    _drain_idx(src2, srcv0, dstv0, isem0)
                    _process(tabs, srcv0, dstv0)

                    @pl.when(g + 1 < GFULL)
                    def _half_b():
                        @pl.when(g + 2 < GFULL)
                        def _pf0():
                            _fire_idx(src2, dst2, g + 2, srcv0, dstv0, isem0)
                        _drain_idx(src2, srcv1, dstv1, isem1)
                        _process(tabs, srcv1, dstv1)
            plsc.subcore_barrier()
            pltpu.sync_copy(acc.at[pl.ds(s * RPT, RPT)],
                            out.at[r].at[pl.ds(s * RPT, RPT)])

    return _seg


def _pad_edges(e, half):
    npad = E_PAD - E
    ar = jnp.arange(npad, dtype=jnp.int32)
    src = jnp.concatenate([e[0] * 8 + 4 * half, (ar % 1024) * 8 + 4 * half])
    dst = jnp.concatenate([e[1], N + (ar % 64)])
    return src.reshape(NWIN * NB, IB), dst.reshape(NWIN * NB, IB)


def _segsum2(tcomb, e_a, e_b):
    tab16 = tcomb.reshape(NT16, CW)
    src_a, dst_a = _pad_edges(e_a, 0)
    src_b, dst_b = _pad_edges(e_b, 1)
    lo = _make_seg(0)(tab16, src_a, dst_a, src_b, dst_b)
    hi = _make_seg(2)(tab16, src_a, dst_a, src_b, dst_b)
    return lo, hi


BLK = 4000
_GRID = ((N + BLK - 1) // BLK,)


def _bspec_h(i):
    return (i, 0)


def _bspec_w(i):
    return (0, 0)


def _bspec_g(i):
    return (0, i, 0)


_H_SPEC = pl.BlockSpec((BLK, U), _bspec_h)
_T_SPEC = pl.BlockSpec((BLK, 2 * U), _bspec_h)
_G_SPEC = pl.BlockSpec((2, BLK, CW), _bspec_g)
_B_SPEC = pl.BlockSpec((1, U), _bspec_w)
_TS = jax.ShapeDtypeStruct((NT, 2 * U), jnp.float32)
_HS = jax.ShapeDtypeStruct((N, U), jnp.float32)


def _wspec(k, m=U):
    return pl.BlockSpec((k, m), _bspec_w)


def _dotp(x, w):
    return jnp.dot(x, w, preferred_element_type=jnp.float32)


def _cat(lo_ref, hi_ref):
    return jnp.concatenate(
        [lo_ref[0], lo_ref[1], hi_ref[0], hi_ref[1]], axis=-1)


def _k0_body(h1, h2, h3, h4, wc2, wu2, wu3, wu4, wd1, b2, b3, b4, bd1,
             t2, u2, u3, u4, v1):
    x1 = h1[...]
    t2[...] = _dotp(x1, wc2[...])
    u2[...] = _dotp(h2[...], wu2[...]) + b2[...]
    u3[...] = _dotp(h3[...], wu3[...]) + b3[...]
    u4[...] = _dotp(h4[...], wu4[...]) + b4[...]
    v1[...] = _dotp(x1, wd1[...]) + bd1[...]


def _k0(h1, h2, h3, h4, wc2, wu2, wu3, wu4, wd1, b2, b3, b4, bd1):
    return pl.pallas_call(
        _k0_body,
        grid=_GRID,
        in_specs=[_H_SPEC] * 4 + [_wspec(U, 2 * U)] + [_wspec(U)] * 4
        + [_B_SPEC] * 4,
        out_specs=[_T_SPEC, _H_SPEC, _H_SPEC, _H_SPEC, _H_SPEC],
        out_shape=[_TS, _HS, _HS, _HS, _HS],
    )(h1, h2, h3, h4, wc2, wu2, wu3, wu4, wd1, b2, b3, b4, bd1)


def _ku_body(u, acclo, acchi, wc, wd, bd, t, v):
    hnew = jnp.tanh(u[...] + _cat(acclo, acchi))
    t[...] = _dotp(hnew, wc[...])
    v[...] = _dotp(hnew, wd[...]) + bd[...]


def _ku(u, acc, wc, wd, bd):
    return pl.pallas_call(
        _ku_body,
        grid=_GRID,
        in_specs=[_H_SPEC, _G_SPEC, _G_SPEC, _wspec(U, 2 * U), _wspec(U),
                  _B_SPEC],
        out_specs=[_T_SPEC, _H_SPEC],
        out_shape=[_TS, _HS],
    )(u, acc[0], acc[1], wc, wd, bd)


def _kp_body(u, acclo, acchi, wc, hf, t):
    h = jnp.tanh(u[...] + _cat(acclo, acchi))
    hf[...] = h
    t[...] = _dotp(h, wc[...])


def _kp(u, acc, wc):
    return pl.pallas_call(
        _kp_body,
        grid=_GRID,
        in_specs=[_H_SPEC, _G_SPEC, _G_SPEC, _wspec(U, 2 * U)],
        out_specs=[_H_SPEC, _T_SPEC],
        out_shape=[_HS, _TS],
    )(u, acc[0], acc[1], wc)


def _kf_body(v, acclo, acchi, hf):
    hf[...] = jnp.tanh(v[...] + _cat(acclo, acchi))


def _kf(v, acc):
    return pl.pallas_call(
        _kf_body,
        grid=_GRID,
        in_specs=[_H_SPEC, _G_SPEC, _G_SPEC],
        out_specs=_H_SPEC,
        out_shape=_HS,
    )(v, acc[0], acc[1])


def kernel(h1, h2, h3, h4, up_2_0, down_2_0, up_2_1, down_2_1, up_3_0, down_3_0, up_3_1, down_3_1, up_4_0, down_4_0, up_4_1, down_4_1, W_up_2, b_up_2, W_up_3, b_up_3, W_up_4, b_up_4, W_down_1, b_down_1, W_down_2, b_down_2, W_down_3, b_down_3):
    wu2, wu3, wu4 = W_up_2.T, W_up_3.T, W_up_4.T
    wd1, wd2, wd3 = W_down_1.T, W_down_2.T, W_down_3.T
    wc2 = jnp.concatenate([wu2[U:2 * U, :], wu2[2 * U:3 * U, :]], axis=1)
    wc3 = jnp.concatenate([wu3[U:2 * U, :], wu3[2 * U:3 * U, :]], axis=1)
    wc4 = jnp.concatenate([wu4[U:2 * U, :], wu4[2 * U:3 * U, :]], axis=1)
    wcd3 = jnp.concatenate([wd3[U:2 * U, :], wd3[U:2 * U, :]], axis=1)
    wcd2 = jnp.concatenate([wd2[U:2 * U, :], wd2[U:2 * U, :]], axis=1)
    wcd1 = jnp.concatenate([wd1[U:2 * U, :], wd1[U:2 * U, :]], axis=1)
    b2, b3, b4 = (b.reshape(1, U) for b in (b_up_2, b_up_3, b_up_4))
    bd1, bd2, bd3 = (b.reshape(1, U) for b in (b_down_1, b_down_2, b_down_3))

    t2, u2, u3, u4, v1 = _k0(h1, h2, h3, h4, wc2, wu2[0:U, :], wu3[0:U, :],
                             wu4[0:U, :], wd1[0:U, :], b2, b3, b4, bd1)
    acc2 = _segsum2(t2, up_2_0, up_2_1)
    t3, v2 = _ku(u2, acc2, wc3, wd2[0:U, :], bd2)
    acc3 = _segsum2(t3, up_3_0, up_3_1)
    t4, v3 = _ku(u3, acc3, wc4, wd3[0:U, :], bd3)
    acc4 = _segsum2(t4, up_4_0, up_4_1)
    h4f, td4 = _kp(u4, acc4, wcd3)
    accd4 = _segsum2(td4, down_4_0, down_4_1)
    h3f, td3 = _kp(v3, accd4, wcd2)
    accd3 = _segsum2(td3, down_3_0, down_3_1)
    h2f, td2 = _kp(v2, accd3, wcd1)
    accd2 = _segsum2(td2, down_2_0, down_2_1)
    h1f = _kf(v1, accd2)
    return jnp.stack([h1f, h2f, h3f, h4f])
